# Initial kernel scaffold; baseline (speedup 1.0000x reference)
import jax
import jax.numpy as jnp
from jax.experimental import pallas as pl


def kernel(x, xe, tables):
    raise NotImplementedError("write your pallas kernel here")


# speedup vs baseline: 1.1055x; 1.1055x over previous
import jax
import jax.numpy as jnp
from jax import lax
from jax.experimental import pallas as pl
from jax.experimental.pallas import tpu as pltpu
from jax.experimental.pallas import tpu_sc as plsc

BATCH = 16384
NUM_CONT = 13
NUM_ENUM = 26
VOCAB = 100000
EMB = 16
OUT_W = NUM_CONT + NUM_ENUM * EMB

NC = 2
NS = 16
NW = NC * NS
ROWS_PER_W = BATCH // NW
CHUNK = 64
N_CHUNK = ROWS_PER_W // CHUNK


def _body(x_hbm, xet_hbm, tab_hbm, out_hbm, idxbuf, xbuf, gbuf, outv, sem):
    wid = lax.axis_index("s") * NC + lax.axis_index("c")
    base = wid * ROWS_PER_W

    pltpu.sync_copy(xet_hbm.at[:, pl.ds(base, ROWS_PER_W)], idxbuf)

    def chunk_body(c, carry):
        row0 = base + c * CHUNK
        pltpu.sync_copy(x_hbm.at[pl.ds(row0 * NUM_CONT, CHUNK * NUM_CONT)],
                        xbuf.at[pl.ds(0, CHUNK * NUM_CONT)])

        copies = [
            pltpu.make_async_copy(
                tab_hbm.at[i].at[idxbuf.at[i, pl.ds(c * CHUNK, CHUNK)]],
                gbuf.at[pl.ds(i * CHUNK, CHUNK)], sem)
            for i in range(NUM_ENUM)
        ]
        for cp in copies:
            cp.start()
        for cp in copies:
            cp.wait()

        def row_body(r, rc):
            outv[r, pl.ds(0, 16)] = xbuf[pl.ds(r * NUM_CONT, 16)]
            for i in range(NUM_ENUM):
                outv[r, pl.ds(NUM_CONT + i * EMB, EMB)] = gbuf[i * CHUNK + r, :]
            return rc

        lax.fori_loop(0, CHUNK, row_body, 0)
        pltpu.sync_copy(outv, out_hbm.at[pl.ds(row0, CHUNK)])
        return carry

    lax.fori_loop(0, N_CHUNK, chunk_body, 0)


@jax.jit
def kernel(x, xe, tables):
    mesh = plsc.VectorSubcoreMesh(core_axis_name="c", subcore_axis_name="s")
    run = pl.kernel(
        _body, mesh=mesh,
        out_type=jax.ShapeDtypeStruct((BATCH, OUT_W), jnp.float32),
        scratch_types=[
            pltpu.VMEM((NUM_ENUM, ROWS_PER_W), jnp.int32),
            pltpu.VMEM((CHUNK * NUM_CONT + 16,), jnp.float32),
            pltpu.VMEM((NUM_ENUM * CHUNK, EMB), jnp.float32),
            pltpu.VMEM((CHUNK, OUT_W), jnp.float32),
            pltpu.SemaphoreType.DMA,
        ],
        compiler_params=pltpu.CompilerParams(use_tc_tiling_on_sc=False),
    )
    return run(x.reshape(-1), xe.T, tables)
